# baseline (speedup 1.0000x reference)
import jax
import jax.numpy as jnp
from jax import lax
from jax.experimental import pallas as pl
from jax.experimental.pallas import tpu as pltpu
from jax.experimental.pallas import tpu_sc as plsc

_NUM_CLASSES = 100
_S = 30.0
_BLKC = 2048

_SC_WORKERS = 32
_SC_LANES = 16
_PAD_C = 112


def _margin_gather_sc(target, bm_pad, w_pad, batch):
    per_w = batch // _SC_WORKERS
    mesh = plsc.VectorSubcoreMesh(core_axis_name="c", subcore_axis_name="s")

    @pl.kernel(
        out_type=jax.ShapeDtypeStruct((batch,), jnp.float32),
        mesh=mesh,
        compiler_params=pltpu.CompilerParams(needs_layout_passes=False),
        scratch_types=[
            pltpu.VMEM((per_w,), jnp.int32),
            pltpu.VMEM((_PAD_C,), jnp.float32),
            pltpu.VMEM((_PAD_C,), jnp.float32),
            pltpu.VMEM((_PAD_C,), jnp.float32),
            pltpu.VMEM((per_w,), jnp.float32),
        ],
    )
    def sc_kernel(t_hbm, bm_hbm, w_hbm, out_hbm, t_v, bm_v, w_v, m_v, out_v):
        wid = lax.axis_index("s") * 2 + lax.axis_index("c")
        base = wid * per_w
        pltpu.sync_copy(t_hbm.at[pl.ds(base, per_w)], t_v)
        pltpu.sync_copy(bm_hbm, bm_v)
        pltpu.sync_copy(w_hbm, w_v)

        @pl.loop(0, _PAD_C, step=_SC_LANES)
        def _(j):
            wv = w_v[pl.ds(j, _SC_LANES)]
            sig = 1.0 / (1.0 + jnp.exp(-wv))
            m_v[pl.ds(j, _SC_LANES)] = bm_v[pl.ds(j, _SC_LANES)] * sig

        @pl.loop(0, per_w, step=_SC_LANES)
        def _(j):
            idx = t_v[pl.ds(j, _SC_LANES)]
            out_v[pl.ds(j, _SC_LANES)] = plsc.load_gather(m_v, [idx])

        pltpu.sync_copy(out_v, out_hbm.at[pl.ds(base, per_w)])

    return sc_kernel(target, bm_pad, w_pad)


def _ldam_body(logit_ref, tgt_ref, mcol_ref, out_ref):
    i = pl.program_id(0)
    n = pl.num_programs(0)
    x = logit_ref[...]
    t = tgt_ref[...]
    m_col = mcol_ref[...]

    cls = jax.lax.broadcasted_iota(jnp.int32, x.shape, 0)
    onehot = cls == t
    picked = jnp.sum(jnp.where(onehot, x, 0.0), axis=0, keepdims=True)
    adj = picked - _S * m_col

    mx = jnp.max(x, axis=0, keepdims=True)
    z = jnp.sum(jnp.exp(x - mx), axis=0, keepdims=True)
    zp = z - jnp.exp(picked - mx) + jnp.exp(adj - mx)
    loss = mx + jnp.log(zp) - adj
    s = jnp.sum(loss, axis=1, keepdims=True)

    @pl.when(i == 0)
    def _():
        out_ref[...] = jnp.zeros_like(out_ref)

    out_ref[...] += s

    @pl.when(i == n - 1)
    def _():
        out_ref[...] = out_ref[...] / (n * _BLKC)


def kernel(logit, target, base_m_list, class_margin_weights):
    b, c = logit.shape
    bm_pad = jnp.pad(base_m_list, (0, _PAD_C - c))
    w_pad = jnp.pad(class_margin_weights, (0, _PAD_C - c))
    m_g = _margin_gather_sc(target, bm_pad, w_pad, b)

    xt = logit.T
    tgt2 = target.reshape(1, b)
    mg2 = m_g.reshape(1, b)
    grid = b // _BLKC
    out = pl.pallas_call(
        _ldam_body,
        grid=(grid,),
        in_specs=[
            pl.BlockSpec((c, _BLKC), lambda i: (0, i)),
            pl.BlockSpec((1, _BLKC), lambda i: (0, i)),
            pl.BlockSpec((1, _BLKC), lambda i: (0, i)),
        ],
        out_specs=pl.BlockSpec((1, 1), lambda i: (0, 0)),
        out_shape=jax.ShapeDtypeStruct((1, 1), jnp.float32),
    )(xt, tgt2, mg2)
    return out[0, 0]


# speedup vs baseline: 1.0528x; 1.0528x over previous
import jax
import jax.numpy as jnp
from jax import lax
from jax.experimental import pallas as pl
from jax.experimental.pallas import tpu as pltpu
from jax.experimental.pallas import tpu_sc as plsc

_NUM_CLASSES = 100
_S = 30.0
_BLKC = 2048

_SC_WORKERS = 32
_SC_LANES = 16
_PAD_C = 112


def _margin_gather_sc(tgt2, base_m_list, class_margin_weights, batch):
    per_w = batch // _SC_WORKERS
    c = base_m_list.shape[0]
    mesh = plsc.VectorSubcoreMesh(core_axis_name="c", subcore_axis_name="s")

    @pl.kernel(
        out_type=jax.ShapeDtypeStruct((1, batch), jnp.float32),
        mesh=mesh,
        compiler_params=pltpu.CompilerParams(needs_layout_passes=False),
        scratch_types=[
            pltpu.VMEM((per_w,), jnp.int32),
            pltpu.VMEM((_PAD_C,), jnp.float32),
            pltpu.VMEM((_PAD_C,), jnp.float32),
            pltpu.VMEM((_PAD_C,), jnp.float32),
            pltpu.VMEM((per_w,), jnp.float32),
        ],
    )
    def sc_kernel(t_hbm, bm_hbm, w_hbm, out_hbm, t_v, bm_v, w_v, m_v, out_v):
        wid = lax.axis_index("s") * 2 + lax.axis_index("c")
        base = wid * per_w
        pltpu.sync_copy(t_hbm.at[0, pl.ds(base, per_w)], t_v)
        pltpu.sync_copy(bm_hbm, bm_v.at[pl.ds(0, c)])
        pltpu.sync_copy(w_hbm, w_v.at[pl.ds(0, c)])

        @pl.loop(0, _PAD_C, step=_SC_LANES)
        def _(j):
            wv = w_v[pl.ds(j, _SC_LANES)]
            sig = 1.0 / (1.0 + jnp.exp(-wv))
            m_v[pl.ds(j, _SC_LANES)] = bm_v[pl.ds(j, _SC_LANES)] * sig

        @pl.loop(0, per_w, step=_SC_LANES)
        def _(j):
            idx = t_v[pl.ds(j, _SC_LANES)]
            out_v[pl.ds(j, _SC_LANES)] = plsc.load_gather(m_v, [idx])

        pltpu.sync_copy(out_v, out_hbm.at[0, pl.ds(base, per_w)])

    return sc_kernel(tgt2, base_m_list, class_margin_weights)


def _stats_body(logit_ref, tgt_ref, stats_ref):
    x = logit_ref[...]
    t = tgt_ref[...]

    cls = jax.lax.broadcasted_iota(jnp.int32, x.shape, 0)
    onehot = cls == t
    picked = jnp.sum(jnp.where(onehot, x, 0.0), axis=0, keepdims=True)
    mx = jnp.max(x, axis=0, keepdims=True)
    z = jnp.sum(jnp.exp(x - mx), axis=0, keepdims=True)
    stats_ref[...] = jnp.concatenate([mx, z, picked], axis=0)


def _combine_body(stats_ref, mcol_ref, out_ref):
    i = pl.program_id(0)
    n = pl.num_programs(0)
    st = stats_ref[...]
    mx = st[0:1, :]
    z = st[1:2, :]
    picked = st[2:3, :]
    adj = picked - _S * mcol_ref[...]
    zp = z - jnp.exp(picked - mx) + jnp.exp(adj - mx)
    loss = mx + jnp.log(zp) - adj
    s = jnp.sum(loss, axis=1, keepdims=True)

    @pl.when(i == 0)
    def _():
        out_ref[...] = jnp.zeros_like(out_ref)

    out_ref[...] += s

    @pl.when(i == n - 1)
    def _():
        out_ref[...] = out_ref[...] / (n * _BLKC)


def kernel(logit, target, base_m_list, class_margin_weights):
    b, c = logit.shape
    tgt2 = target.reshape(1, b)
    m_g = _margin_gather_sc(tgt2, base_m_list, class_margin_weights, b)

    xt = logit.T
    grid = b // _BLKC
    stats = pl.pallas_call(
        _stats_body,
        grid=(grid,),
        in_specs=[
            pl.BlockSpec((c, _BLKC), lambda i: (0, i)),
            pl.BlockSpec((1, _BLKC), lambda i: (0, i)),
        ],
        out_specs=pl.BlockSpec((3, _BLKC), lambda i: (0, i)),
        out_shape=jax.ShapeDtypeStruct((3, b), jnp.float32),
    )(xt, tgt2)

    out = pl.pallas_call(
        _combine_body,
        grid=(grid,),
        in_specs=[
            pl.BlockSpec((3, _BLKC), lambda i: (0, i)),
            pl.BlockSpec((1, _BLKC), lambda i: (0, i)),
        ],
        out_specs=pl.BlockSpec((1, 1), lambda i: (0, 0)),
        out_shape=jax.ShapeDtypeStruct((1, 1), jnp.float32),
    )(stats, m_g)
    return out[0, 0]


# speedup vs baseline: 1.2053x; 1.1448x over previous
import jax
import jax.numpy as jnp
from jax import lax
from jax.experimental import pallas as pl
from jax.experimental.pallas import tpu as pltpu
from jax.experimental.pallas import tpu_sc as plsc

_NUM_CLASSES = 100
_S = 30.0
_BLKC = 2048

_SC_WORKERS = 32
_SC_LANES = 16
_PAD_C = 112


def _margin_gather_sc(tgt2, base_m_list, class_margin_weights, batch):
    per_w = batch // _SC_WORKERS
    c = base_m_list.shape[0]
    mesh = plsc.VectorSubcoreMesh(core_axis_name="c", subcore_axis_name="s")

    @pl.kernel(
        out_type=jax.ShapeDtypeStruct((1, batch), jnp.float32),
        mesh=mesh,
        compiler_params=pltpu.CompilerParams(needs_layout_passes=False),
        scratch_types=[
            pltpu.VMEM((per_w,), jnp.int32),
            pltpu.VMEM((_PAD_C,), jnp.float32),
            pltpu.VMEM((_PAD_C,), jnp.float32),
            pltpu.VMEM((_PAD_C,), jnp.float32),
            pltpu.VMEM((per_w,), jnp.float32),
        ],
    )
    def sc_kernel(t_hbm, bm_hbm, w_hbm, out_hbm, t_v, bm_v, w_v, m_v, out_v):
        wid = lax.axis_index("s") * 2 + lax.axis_index("c")
        base = wid * per_w
        pltpu.sync_copy(t_hbm.at[0, pl.ds(base, per_w)], t_v)
        pltpu.sync_copy(bm_hbm, bm_v.at[pl.ds(0, c)])
        pltpu.sync_copy(w_hbm, w_v.at[pl.ds(0, c)])

        @pl.loop(0, _PAD_C, step=_SC_LANES)
        def _(j):
            wv = w_v[pl.ds(j, _SC_LANES)]
            sig = 1.0 / (1.0 + jnp.exp(-wv))
            m_v[pl.ds(j, _SC_LANES)] = bm_v[pl.ds(j, _SC_LANES)] * sig

        @pl.loop(0, per_w, step=_SC_LANES)
        def _(j):
            idx = t_v[pl.ds(j, _SC_LANES)]
            out_v[pl.ds(j, _SC_LANES)] = plsc.load_gather(m_v, [idx])

        pltpu.sync_copy(out_v, out_hbm.at[0, pl.ds(base, per_w)])

    return sc_kernel(tgt2, base_m_list, class_margin_weights)


def _stats_body(logit_ref, tgt_ref, stats_ref):
    x = logit_ref[...]
    t = tgt_ref[...]

    cls = jax.lax.broadcasted_iota(jnp.int32, x.shape, 0)
    onehot = cls == t
    picked = jnp.sum(jnp.where(onehot, x, 0.0), axis=0, keepdims=True)
    mx = jnp.max(x, axis=0, keepdims=True)
    z = jnp.sum(jnp.exp(x - mx), axis=0, keepdims=True)
    stats_ref[...] = jnp.concatenate([z, picked - mx], axis=0)


def _combine_body(stats_ref, mcol_ref, out_ref):
    st = stats_ref[...]
    z = st[0:1, :]
    u = st[1:2, :]
    sm = _S * mcol_ref[...]
    zp = z - jnp.exp(u) + jnp.exp(u - sm)
    loss = jnp.log(zp) - u + sm
    out_ref[...] = jnp.sum(loss, axis=1, keepdims=True) / loss.shape[1]


def kernel(logit, target, base_m_list, class_margin_weights):
    b, c = logit.shape
    tgt2 = target.reshape(1, b)
    m_g = _margin_gather_sc(tgt2, base_m_list, class_margin_weights, b)

    xt = logit.T
    grid = b // _BLKC
    stats = pl.pallas_call(
        _stats_body,
        grid=(grid,),
        in_specs=[
            pl.BlockSpec((c, _BLKC), lambda i: (0, i)),
            pl.BlockSpec((1, _BLKC), lambda i: (0, i)),
        ],
        out_specs=pl.BlockSpec((2, _BLKC), lambda i: (0, i)),
        out_shape=jax.ShapeDtypeStruct((2, b), jnp.float32),
    )(xt, tgt2)

    out = pl.pallas_call(
        _combine_body,
        out_shape=jax.ShapeDtypeStruct((1, 1), jnp.float32),
    )(stats, m_g)
    return out[0, 0]


# speedup vs baseline: 1.2360x; 1.0255x over previous
import jax
import jax.numpy as jnp
from jax import lax
from jax.experimental import pallas as pl
from jax.experimental.pallas import tpu as pltpu
from jax.experimental.pallas import tpu_sc as plsc

_NUM_CLASSES = 100
_S = 30.0
_BLKC = 4096

_SC_WORKERS = 32
_SC_LANES = 16
_PAD_C = 112


def _margin_gather_sc(tgt2, base_m_list, class_margin_weights, batch):
    per_w = batch // _SC_WORKERS
    c = base_m_list.shape[0]
    mesh = plsc.VectorSubcoreMesh(core_axis_name="c", subcore_axis_name="s")

    @pl.kernel(
        out_type=jax.ShapeDtypeStruct((1, batch), jnp.float32),
        mesh=mesh,
        compiler_params=pltpu.CompilerParams(needs_layout_passes=False),
        scratch_types=[
            pltpu.VMEM((per_w,), jnp.int32),
            pltpu.VMEM((_PAD_C,), jnp.float32),
            pltpu.VMEM((_PAD_C,), jnp.float32),
            pltpu.VMEM((_PAD_C,), jnp.float32),
            pltpu.VMEM((per_w,), jnp.float32),
        ],
    )
    def sc_kernel(t_hbm, bm_hbm, w_hbm, out_hbm, t_v, bm_v, w_v, m_v, out_v):
        wid = lax.axis_index("s") * 2 + lax.axis_index("c")
        base = wid * per_w
        pltpu.sync_copy(t_hbm.at[0, pl.ds(base, per_w)], t_v)
        pltpu.sync_copy(bm_hbm, bm_v.at[pl.ds(0, c)])
        pltpu.sync_copy(w_hbm, w_v.at[pl.ds(0, c)])

        @pl.loop(0, _PAD_C, step=_SC_LANES)
        def _(j):
            wv = w_v[pl.ds(j, _SC_LANES)]
            sig = 1.0 / (1.0 + jnp.exp(-wv))
            m_v[pl.ds(j, _SC_LANES)] = bm_v[pl.ds(j, _SC_LANES)] * sig

        @pl.loop(0, per_w, step=_SC_LANES)
        def _(j):
            idx = t_v[pl.ds(j, _SC_LANES)]
            out_v[pl.ds(j, _SC_LANES)] = plsc.load_gather(m_v, [idx])

        pltpu.sync_copy(out_v, out_hbm.at[0, pl.ds(base, per_w)])

    return sc_kernel(tgt2, base_m_list, class_margin_weights)


def _stats_body(logit_ref, tgt_ref, stats_ref):
    x = logit_ref[...]
    t = tgt_ref[...]

    cls = jax.lax.broadcasted_iota(jnp.int32, x.shape, 0)
    onehot = cls == t
    picked = jnp.sum(jnp.where(onehot, x, 0.0), axis=0, keepdims=True)
    mx = jnp.max(x, axis=0, keepdims=True)
    z = jnp.sum(jnp.exp(x - mx), axis=0, keepdims=True)
    stats_ref[...] = jnp.concatenate([z, picked - mx], axis=0)


def _combine_body(stats_ref, mcol_ref, out_ref):
    st = stats_ref[...]
    z = st[0:1, :]
    u = st[1:2, :]
    sm = _S * mcol_ref[...]
    zp = z - jnp.exp(u) + jnp.exp(u - sm)
    loss = jnp.log(zp) - u + sm
    out_ref[...] = jnp.sum(loss, axis=1, keepdims=True) / loss.shape[1]


def kernel(logit, target, base_m_list, class_margin_weights):
    b, c = logit.shape
    tgt2 = target.reshape(1, b)
    m_g = _margin_gather_sc(tgt2, base_m_list, class_margin_weights, b)

    xt = logit.T
    grid = b // _BLKC
    stats = pl.pallas_call(
        _stats_body,
        grid=(grid,),
        in_specs=[
            pl.BlockSpec((c, _BLKC), lambda i: (0, i)),
            pl.BlockSpec((1, _BLKC), lambda i: (0, i)),
        ],
        out_specs=pl.BlockSpec((2, _BLKC), lambda i: (0, i)),
        out_shape=jax.ShapeDtypeStruct((2, b), jnp.float32),
    )(xt, tgt2)

    out = pl.pallas_call(
        _combine_body,
        out_shape=jax.ShapeDtypeStruct((1, 1), jnp.float32),
    )(stats, m_g)
    return out[0, 0]


# speedup vs baseline: 1.2386x; 1.0021x over previous
import jax
import jax.numpy as jnp
from jax import lax
from jax.experimental import pallas as pl
from jax.experimental.pallas import tpu as pltpu
from jax.experimental.pallas import tpu_sc as plsc

_NUM_CLASSES = 100
_S = 30.0
_BLKC = 8192

_SC_WORKERS = 32
_SC_LANES = 16
_PAD_C = 112


def _margin_gather_sc(tgt2, base_m_list, class_margin_weights, batch):
    per_w = batch // _SC_WORKERS
    c = base_m_list.shape[0]
    mesh = plsc.VectorSubcoreMesh(core_axis_name="c", subcore_axis_name="s")

    @pl.kernel(
        out_type=jax.ShapeDtypeStruct((1, batch), jnp.float32),
        mesh=mesh,
        compiler_params=pltpu.CompilerParams(needs_layout_passes=False),
        scratch_types=[
            pltpu.VMEM((per_w,), jnp.int32),
            pltpu.VMEM((_PAD_C,), jnp.float32),
            pltpu.VMEM((_PAD_C,), jnp.float32),
            pltpu.VMEM((_PAD_C,), jnp.float32),
            pltpu.VMEM((per_w,), jnp.float32),
        ],
    )
    def sc_kernel(t_hbm, bm_hbm, w_hbm, out_hbm, t_v, bm_v, w_v, m_v, out_v):
        wid = lax.axis_index("s") * 2 + lax.axis_index("c")
        base = wid * per_w
        pltpu.sync_copy(t_hbm.at[0, pl.ds(base, per_w)], t_v)
        pltpu.sync_copy(bm_hbm, bm_v.at[pl.ds(0, c)])
        pltpu.sync_copy(w_hbm, w_v.at[pl.ds(0, c)])

        @pl.loop(0, _PAD_C, step=_SC_LANES)
        def _(j):
            wv = w_v[pl.ds(j, _SC_LANES)]
            sig = 1.0 / (1.0 + jnp.exp(-wv))
            m_v[pl.ds(j, _SC_LANES)] = bm_v[pl.ds(j, _SC_LANES)] * sig

        @pl.loop(0, per_w, step=_SC_LANES)
        def _(j):
            idx = t_v[pl.ds(j, _SC_LANES)]
            out_v[pl.ds(j, _SC_LANES)] = plsc.load_gather(m_v, [idx])

        pltpu.sync_copy(out_v, out_hbm.at[0, pl.ds(base, per_w)])

    return sc_kernel(tgt2, base_m_list, class_margin_weights)


def _stats_body(logit_ref, tgt_ref, stats_ref):
    x = logit_ref[...]
    t = tgt_ref[...]

    cls = jax.lax.broadcasted_iota(jnp.int32, x.shape, 0)
    onehot = cls == t
    picked = jnp.sum(jnp.where(onehot, x, 0.0), axis=0, keepdims=True)
    mx = jnp.max(x, axis=0, keepdims=True)
    z = jnp.sum(jnp.exp(x - mx), axis=0, keepdims=True)
    stats_ref[...] = jnp.concatenate([z, picked - mx], axis=0)


def _combine_body(stats_ref, mcol_ref, out_ref):
    st = stats_ref[...]
    z = st[0:1, :]
    u = st[1:2, :]
    sm = _S * mcol_ref[...]
    zp = z - jnp.exp(u) + jnp.exp(u - sm)
    loss = jnp.log(zp) - u + sm
    out_ref[...] = jnp.sum(loss, axis=1, keepdims=True) / loss.shape[1]


def kernel(logit, target, base_m_list, class_margin_weights):
    b, c = logit.shape
    tgt2 = target.reshape(1, b)
    m_g = _margin_gather_sc(tgt2, base_m_list, class_margin_weights, b)

    xt = logit.T
    grid = b // _BLKC
    stats = pl.pallas_call(
        _stats_body,
        grid=(grid,),
        in_specs=[
            pl.BlockSpec((c, _BLKC), lambda i: (0, i)),
            pl.BlockSpec((1, _BLKC), lambda i: (0, i)),
        ],
        out_specs=pl.BlockSpec((2, _BLKC), lambda i: (0, i)),
        out_shape=jax.ShapeDtypeStruct((2, b), jnp.float32),
    )(xt, tgt2)

    out = pl.pallas_call(
        _combine_body,
        out_shape=jax.ShapeDtypeStruct((1, 1), jnp.float32),
    )(stats, m_g)
    return out[0, 0]
